# baseline (speedup 1.0000x reference)
import jax
import jax.numpy as jnp
from jax import lax
from jax.experimental import pallas as pl
from jax.experimental.pallas import tpu as pltpu
from jax.experimental.pallas import tpu_sc as plsc

NUM_CORES = 2
NUM_SUBCORES = 16
NW = NUM_CORES * NUM_SUBCORES
LANES = 16

BATCH = 16384
K = 32
NROWS = 1000000
B_PER_W = BATCH // NW
NQ = B_PER_W // 128


def _sc_kernel(uidx_hbm, iidx_hbm, uflat_hbm, iflat_hbm, out_hbm,
               uidx_v, iidx_v, gidx_u, gidx_i, u_t, v_t, out_v, sem):
    wid = lax.axis_index("s") * NUM_CORES + lax.axis_index("c")

    pltpu.sync_copy(uidx_hbm.at[wid], uidx_v)
    pltpu.sync_copy(iidx_hbm.at[wid], iidx_v)

    @pl.loop(0, NQ)
    def _(q):
        @pl.loop(0, 128 // LANES)
        def _(j):
            ds = pl.ds(j * LANES, LANES)
            ivu = uidx_v[q, ds]
            ivi = iidx_v[q, ds]
            for k in range(K):
                gidx_u[k, q, ds] = ivu + k * NROWS
                gidx_i[k, q, ds] = ivi + k * NROWS

    @pl.loop(0, NQ)
    def _(q):
        cols = pl.ds(q * 128, 128)
        for k in range(K):
            pltpu.async_copy(uflat_hbm.at[gidx_u.at[k, q]],
                             u_t.at[k, cols], sem)
            pltpu.async_copy(iflat_hbm.at[gidx_i.at[k, q]],
                             v_t.at[k, cols], sem)

    @pl.loop(0, NQ)
    def _(q):
        cols = pl.ds(q * 128, 128)
        for k in range(K):
            pltpu.make_async_copy(uflat_hbm.at[gidx_u.at[k, q]],
                                  u_t.at[k, cols], sem).wait()
            pltpu.make_async_copy(iflat_hbm.at[gidx_i.at[k, q]],
                                  v_t.at[k, cols], sem).wait()

    zero = jnp.zeros((LANES,), jnp.float32)

    @pl.loop(0, B_PER_W // LANES)
    def _(c):
        cols = pl.ds(c * LANES, LANES)
        acc = zero
        for k in range(K):
            u = jnp.maximum(u_t[k, cols], zero)
            v = jnp.maximum(v_t[k, cols], zero)
            acc = acc + u * v
        out_v[cols] = acc

    pltpu.sync_copy(out_v, out_hbm.at[wid])


@jax.jit
def kernel(user_idx, item_idx, user_emb, item_emb):
    uidx = user_idx.reshape(NW, NQ, 128)
    iidx = item_idx.reshape(NW, NQ, 128)
    uflat = user_emb.T.reshape(K * NROWS)
    iflat = item_emb.T.reshape(K * NROWS)
    mesh = plsc.VectorSubcoreMesh(core_axis_name="c", subcore_axis_name="s")
    cp = pltpu.CompilerParams(needs_layout_passes=False,
                              use_tc_tiling_on_sc=False)
    run = pl.kernel(
        _sc_kernel,
        out_type=jax.ShapeDtypeStruct((NW, B_PER_W), jnp.float32),
        mesh=mesh,
        scratch_types=[
            pltpu.VMEM((NQ, 128), jnp.int32),
            pltpu.VMEM((NQ, 128), jnp.int32),
            pltpu.VMEM((K, NQ, 128), jnp.int32),
            pltpu.VMEM((K, NQ, 128), jnp.int32),
            pltpu.VMEM((K, B_PER_W), jnp.float32),
            pltpu.VMEM((K, B_PER_W), jnp.float32),
            pltpu.VMEM((B_PER_W,), jnp.float32),
            pltpu.SemaphoreType.DMA,
        ],
        compiler_params=cp,
    )
    out = run(uidx, iidx, uflat, iflat)
    return out.reshape(BATCH)
